# Initial kernel scaffold; baseline (speedup 1.0000x reference)
import jax
import jax.numpy as jnp
from jax.experimental import pallas as pl


def kernel(h, h_bond, pos, bond_index, hk_W1, hk_b1, hk_g, hk_beta, hk_W2, hk_b2, hv_W1, hv_b1, hv_g, hv_beta, hv_W2, hv_b2, hq_W1, hq_b1, hq_g, hq_beta, hq_W2, hq_b2):
    raise NotImplementedError("write your pallas kernel here")


# speedup vs baseline: 18.1147x; 18.1147x over previous
import functools
import math

import jax
import jax.numpy as jnp
import numpy as np
from jax.experimental import pallas as pl
from jax.experimental.pallas import tpu as pltpu

N_HEADS = 16
HEAD_DIM = 8
NUM_GAUSSIANS = 20
DIM = 128
_GS_SPACING = 10.0 / (NUM_GAUSSIANS - 1)
_GS_COEFF = -0.5 / _GS_SPACING ** 2
_FREQS = np.array([[1.0, 2.0, 3.0, 1.0, 0.5, 1.0 / 3.0]], dtype=np.float32)
_NEG_INF = float("-inf")


def _ln_relu(x, g, beta):
    mu = jnp.mean(x, axis=-1, keepdims=True)
    var = jnp.mean((x - mu) ** 2, axis=-1, keepdims=True)
    y = (x - mu) * jax.lax.rsqrt(var + 1e-5) * g + beta
    return jnp.maximum(y, 0.0)


def _dot(a, b):
    return jnp.dot(a, b, preferred_element_type=jnp.float32)


def _precompute_kernel(hb_ref, posi_ref, posj_ref,
                       w1q_ref, b1q_ref, gq_ref, betaq_ref, w2q_ref, b2q_ref,
                       w1ka_ref, w1kb_ref, w1kc_ref, b1k_ref,
                       w1va_ref, w1vb_ref, w1vc_ref, b1v_ref,
                       q_ref, uk_ref, uv_ref, vk_ref, vv_ref):
    hb = hb_ref[...]
    posi = posi_ref[...]
    posj = posj_ref[...]
    d = posi - posj
    dist = jnp.sqrt(jnp.sum(d * d, axis=-1, keepdims=True))
    offs = jax.lax.broadcasted_iota(
        jnp.int32, (hb.shape[0], NUM_GAUSSIANS), 1).astype(jnp.float32)
    diff = dist - offs * _GS_SPACING
    r_feat = jnp.exp(_GS_COEFF * diff * diff)

    hmid_q = _dot(hb, w1q_ref[...]) + b1q_ref[...]
    rq = _ln_relu(hmid_q, gq_ref[...], betaq_ref[...])
    q_ref[...] = _dot(rq, w2q_ref[...]) + b2q_ref[...]

    uk_ref[...] = _dot(hb, w1ka_ref[...]) + _dot(r_feat, w1kb_ref[...])
    uv_ref[...] = _dot(hb, w1va_ref[...]) + _dot(r_feat, w1vb_ref[...])
    vk_ref[...] = _dot(r_feat, w1kc_ref[...]) + b1k_ref[...]
    vv_ref[...] = _dot(r_feat, w1vc_ref[...]) + b1v_ref[...]


def _slot_kernel(uk2_ref, uv2_ref, posj2_ref, valid_ref,
                 vk_ref, vv_ref, q_ref, posji_ref, posi_ref,
                 freqs_ref,
                 w1kd_ref, gk_ref, betak_ref, w2k_ref,
                 w1vd_ref, gv_ref, betav_ref, w2v_ref, b2v_ref,
                 pool_ref, poolt_ref,
                 m_ref, l_ref, acc_ref,
                 m_out, l_out, acc_out):
    posji = posji_ref[...]
    pos_ki = posj2_ref[...] - posi_ref[...]
    a = jnp.sum(posji * pos_ki, axis=-1, keepdims=True)
    x0, y0, z0 = posji[:, 0:1], posji[:, 1:2], posji[:, 2:3]
    x1, y1, z1 = pos_ki[:, 0:1], pos_ki[:, 1:2], pos_ki[:, 2:3]
    c0 = y0 * z1 - z0 * y1
    c1 = z0 * x1 - x0 * z1
    c2 = x0 * y1 - y0 * x1
    b = jnp.sqrt(c0 * c0 + c1 * c1 + c2 * c2 + 1e-12)
    angle = jnp.arctan2(b, a)
    z = angle * freqs_ref[...][:, 0:6]
    s6 = jnp.sin(z)
    c6 = jnp.cos(z)

    def ang_term(wd_ref):
        wd = wd_ref[...]
        return (angle * wd[0:1, :] + _dot(s6, wd[1:7, :]) + _dot(c6, wd[7:13, :]))

    valid = valid_ref[...] > 0

    hmid_k = uk2_ref[...] + vk_ref[...] + ang_term(w1kd_ref)
    rk = _ln_relu(hmid_k, gk_ref[...], betak_ref[...])
    k2 = _dot(rk, w2k_ref[...])
    logits = _dot(q_ref[...] * k2, pool_ref[...]) * (1.0 / math.sqrt(HEAD_DIM))
    logits_m = jnp.where(valid, logits, _NEG_INF)

    m_old = m_ref[...]
    m_new = jnp.maximum(m_old, logits_m)
    scale = jnp.where(m_new == _NEG_INF, 0.0, jnp.exp(m_old - m_new))
    p16 = jnp.where(valid, jnp.exp(logits - m_new), 0.0)
    m_out[...] = m_new
    l_out[...] = l_ref[...] * scale + p16

    hmid_v = uv2_ref[...] + vv_ref[...] + ang_term(w1vd_ref)
    rv = _ln_relu(hmid_v, gv_ref[...], betav_ref[...])
    v2 = _dot(rv, w2v_ref[...]) + b2v_ref[...]
    poolt = poolt_ref[...]
    scale128 = _dot(scale, poolt)
    p128 = _dot(p16, poolt)
    acc_out[...] = acc_ref[...] * scale128 + p128 * v2


def _finalize_kernel(acc_ref, l_ref, poolt_ref, out_ref):
    l128 = _dot(l_ref[...], poolt_ref[...])
    out_ref[...] = jnp.where(l128 > 0.0, acc_ref[...] / l128, 0.0)


def _edge_spec(be, lanes):
    return pl.BlockSpec((be, lanes), lambda i: (i, 0))


def _full_spec(shape):
    return pl.BlockSpec(shape, lambda i: tuple(0 for _ in shape))


def kernel(h, h_bond, pos, bond_index,
           hk_W1, hk_b1, hk_g, hk_beta, hk_W2, hk_b2,
           hv_W1, hv_b1, hv_g, hv_beta, hv_W2, hv_b2,
           hq_W1, hq_b1, hq_g, hq_beta, hq_W2, hq_b2):
    N = h.shape[0]
    E = h_bond.shape[0]
    BE = 1000 if E % 1000 == 0 else 8
    Ep = ((E + BE - 1) // BE) * BE
    nblk = Ep // BE

    row = bond_index[0].astype(jnp.int32)
    col = bond_index[1].astype(jnp.int32)
    order = jnp.argsort(col, stable=True).astype(jnp.int32)
    counts = jnp.zeros((N,), jnp.int32).at[col].add(1)
    starts = jnp.concatenate([jnp.zeros((1,), jnp.int32), jnp.cumsum(counts, dtype=jnp.int32)])
    j_start = starts[row]
    j_count = counts[row]
    max_cnt = jnp.max(counts)

    posi = pos[col]
    posj = pos[row]

    def padE(x, lanes=None):
        if x.ndim == 1:
            x = x[:, None]
        pad_lanes = 0 if lanes is None else lanes - x.shape[1]
        return jnp.pad(x, ((0, Ep - E), (0, pad_lanes)))

    hb_p = padE(h_bond)
    posi_p = padE(posi, 8)
    posj_p = padE(posj, 8)

    row1 = hq_b1[None, :]
    g1 = hq_g[None, :]
    beta1 = hq_beta[None, :]
    b2q = hq_b2[None, :]
    w1ka, w1kb, w1kc = hk_W1[0:128], hk_W1[128:148], hk_W1[148:168]
    w1va, w1vb, w1vc = hv_W1[0:128], hv_W1[128:148], hv_W1[148:168]
    w1kd = jnp.pad(hk_W1[168:181], ((0, 3), (0, 0)))
    w1vd = jnp.pad(hv_W1[168:181], ((0, 3), (0, 0)))
    b1k = hk_b1[None, :]
    b1v = hv_b1[None, :]

    q_p, uk_p, uv_p, vk_p, vv_p = pl.pallas_call(
        _precompute_kernel,
        grid=(nblk,),
        in_specs=[
            _edge_spec(BE, DIM), _edge_spec(BE, 8), _edge_spec(BE, 8),
            _full_spec((DIM, DIM)), _full_spec((1, DIM)), _full_spec((1, DIM)),
            _full_spec((1, DIM)), _full_spec((DIM, DIM)), _full_spec((1, DIM)),
            _full_spec((DIM, DIM)), _full_spec((NUM_GAUSSIANS, DIM)),
            _full_spec((NUM_GAUSSIANS, DIM)), _full_spec((1, DIM)),
            _full_spec((DIM, DIM)), _full_spec((NUM_GAUSSIANS, DIM)),
            _full_spec((NUM_GAUSSIANS, DIM)), _full_spec((1, DIM)),
        ],
        out_specs=[_edge_spec(BE, DIM)] * 5,
        out_shape=[jax.ShapeDtypeStruct((Ep, DIM), jnp.float32)] * 5,
    )(hb_p, posi_p, posj_p,
      hq_W1, row1, g1, beta1, hq_W2, b2q,
      w1ka, w1kb, w1kc, b1k,
      w1va, w1vb, w1vc, b1v)

    posji_p = posj_p - posi_p

    pool = jnp.asarray(np.kron(np.eye(N_HEADS), np.ones((HEAD_DIM, 1))), jnp.float32)
    poolt = pool.T
    freqs = jnp.asarray(np.pad(_FREQS, ((0, 0), (0, 2))), jnp.float32)

    m0 = jnp.full((Ep, N_HEADS), _NEG_INF, jnp.float32)
    l0 = jnp.zeros((Ep, N_HEADS), jnp.float32)
    acc0 = jnp.zeros((Ep, DIM), jnp.float32)

    gk = hk_g[None, :]
    betak = hk_beta[None, :]
    gv = hv_g[None, :]
    betav = hv_beta[None, :]
    b2v = hv_b2[None, :]

    slot_call = pl.pallas_call(
        _slot_kernel,
        grid=(nblk,),
        in_specs=[
            _edge_spec(BE, DIM), _edge_spec(BE, DIM), _edge_spec(BE, 8),
            _edge_spec(BE, 1),
            _edge_spec(BE, DIM), _edge_spec(BE, DIM), _edge_spec(BE, DIM),
            _edge_spec(BE, 8), _edge_spec(BE, 8),
            _full_spec((1, 8)),
            _full_spec((16, DIM)), _full_spec((1, DIM)), _full_spec((1, DIM)),
            _full_spec((DIM, DIM)),
            _full_spec((16, DIM)), _full_spec((1, DIM)), _full_spec((1, DIM)),
            _full_spec((DIM, DIM)), _full_spec((1, DIM)),
            _full_spec((DIM, N_HEADS)), _full_spec((N_HEADS, DIM)),
            _edge_spec(BE, N_HEADS), _edge_spec(BE, N_HEADS), _edge_spec(BE, DIM),
        ],
        out_specs=[_edge_spec(BE, N_HEADS), _edge_spec(BE, N_HEADS),
                   _edge_spec(BE, DIM)],
        out_shape=[jax.ShapeDtypeStruct((Ep, N_HEADS), jnp.float32),
                   jax.ShapeDtypeStruct((Ep, N_HEADS), jnp.float32),
                   jax.ShapeDtypeStruct((Ep, DIM), jnp.float32)],
        input_output_aliases={21: 0, 22: 1, 23: 2},
    )

    def body(s, state):
        m, l, acc = state
        pidx = jnp.minimum(j_start + s, E - 1)
        e2 = order[pidx]
        valid = ((s < j_count) & (col != row[e2])).astype(jnp.int32)
        uk2 = padE(uk_p[:E][e2])
        uv2 = padE(uv_p[:E][e2])
        posj2 = padE(posj[e2], 8)
        valid_p = padE(valid)
        return slot_call(uk2, uv2, posj2, valid_p,
                         vk_p, vv_p, q_p, posji_p, posi_p,
                         freqs,
                         w1kd, gk, betak, hk_W2,
                         w1vd, gv, betav, hv_W2, b2v,
                         pool, poolt,
                         m, l, acc)

    m, l, acc = jax.lax.fori_loop(0, max_cnt, body, (m0, l0, acc0))

    out = pl.pallas_call(
        _finalize_kernel,
        grid=(nblk,),
        in_specs=[_edge_spec(BE, DIM), _edge_spec(BE, N_HEADS),
                  _full_spec((N_HEADS, DIM))],
        out_specs=_edge_spec(BE, DIM),
        out_shape=jax.ShapeDtypeStruct((Ep, DIM), jnp.float32),
    )(acc, l, poolt)

    return out[:E]
